# Initial kernel scaffold; baseline (speedup 1.0000x reference)
import jax
import jax.numpy as jnp
from jax.experimental import pallas as pl


def kernel(x, edge_index, W0, b0, aW0, ab0, W1, b1, aW1, ab1, W2, b2, aW2, ab2):
    raise NotImplementedError("write your pallas kernel here")


# speedup vs baseline: 6.0024x; 6.0024x over previous
import functools

import jax
import jax.numpy as jnp
from jax import lax
from jax.experimental import pallas as pl
from jax.experimental.pallas import tpu as pltpu
from jax.experimental.pallas import tpu_sc as plsc

N = 10000
E = 320000
D = 128
NP = 10240
NSUB = 16
NCORE = 2
NPHASE = 2
NQUARTER = NCORE * NPHASE
NQ = NP // NQUARTER
ROWS_PER_TILE = NQ // NSUB
ACC_ROWS = NQ + 128
EDGES_PER_TILE = E // NSUB
CHUNK = 128
FULL_CHUNKS = EDGES_PER_TILE // CHUNK
TAIL = EDGES_PER_TILE - FULL_CHUNKS * CHUNK
DEN_R, DEN_C = NP // 16, 16


_BLK = 1024


def _emit_hu(h, awd, ab, hu_ref, u_ref):
    u = jnp.exp(jnp.dot(h, awd, preferred_element_type=jnp.float32) + ab)
    hu_ref[...] = h * u
    u_ref[...] = u


def _normalized_input(acc_ref, den_ref):
    den = den_ref[...]
    return jnp.maximum(acc_ref[...] / jnp.where(den > 0, den, 1.0), 0.0)


def _tc_layer_body(flag_ref, x_ref, acc_ref, den_ref, w_ref, b_ref,
                   awd_ref, ab_ref, hu_ref, u_ref):
    xin = _normalized_input(acc_ref, den_ref)
    xin = jnp.where(flag_ref[0, 0] > 0, x_ref[...], xin)
    h = jnp.dot(xin, w_ref[...], preferred_element_type=jnp.float32)
    h = h + b_ref[...]
    h = jnp.where(h >= 0, h, 0.2 * h)
    _emit_hu(h, awd_ref[...], ab_ref[...], hu_ref, u_ref)


def _tc_final_body(acc_ref, den_ref, o_ref):
    o_ref[...] = _normalized_input(acc_ref, den_ref)


_w_spec = pl.BlockSpec((D, D), lambda i: (0, 0))
_b_spec = pl.BlockSpec((1, D), lambda i: (0, 0))
_awd_spec = pl.BlockSpec((D, 1), lambda i: (0, 0))
_ab_spec = pl.BlockSpec((1, 1), lambda i: (0, 0))
_acc_spec = pl.BlockSpec((_BLK, D), lambda i: (i, 0))
_den_spec = pl.BlockSpec((_BLK, 1), lambda i: (i, 0))
_hu_spec = pl.BlockSpec((_BLK, D), lambda i: (i, 0))
_u_spec = pl.BlockSpec((_BLK, 1), lambda i: (i, 0))

_hu_shapes = (jax.ShapeDtypeStruct((NP, D), jnp.float32),
              jax.ShapeDtypeStruct((NP, 1), jnp.float32))

_tc_layer = pl.pallas_call(
    _tc_layer_body,
    grid=(NP // _BLK,),
    in_specs=[pl.BlockSpec((1, 1), lambda i: (0, 0)),
              pl.BlockSpec((_BLK, D), lambda i: (i, 0)),
              _acc_spec, _den_spec, _w_spec, _b_spec, _awd_spec, _ab_spec],
    out_specs=(_hu_spec, _u_spec),
    out_shape=_hu_shapes,
)

_tc_final = pl.pallas_call(
    _tc_final_body,
    grid=(NP // _BLK,),
    in_specs=[_acc_spec, _den_spec],
    out_specs=pl.BlockSpec((_BLK, D), lambda i: (i, 0)),
    out_shape=jax.ShapeDtypeStruct((NP, D), jnp.float32),
)


_sc_mesh = plsc.VectorSubcoreMesh(core_axis_name="c", subcore_axis_name="s",
                                  num_cores=NCORE)


@functools.partial(
    pl.kernel,
    mesh=_sc_mesh,
    compiler_params=pltpu.CompilerParams(needs_layout_passes=False),
    out_type=(jax.ShapeDtypeStruct((NQUARTER, NQ, D), jnp.float32),
              jax.ShapeDtypeStruct((NCORE, NSUB, DEN_R, DEN_C), jnp.float32)),
    scratch_types=[
        pltpu.VMEM((CHUNK,), jnp.int32),
        pltpu.VMEM((CHUNK,), jnp.int32),
        pltpu.VMEM((CHUNK, D), jnp.float32),
        pltpu.VMEM((NP,), jnp.float32),
        pltpu.VMEM((DEN_R, DEN_C), jnp.float32),
        pltpu.VMEM_SHARED((ACC_ROWS, D), jnp.float32),
        pltpu.SemaphoreType.DMA,
    ],
)
def _sc_segsum(hu_hbm, u_hbm, src_hbm, dst_hbm, zrows_hbm, zden_hbm,
               acc_out, den_out,
               sidx, didx, rows, u_loc, den_loc, acc, gsem):
    c = lax.axis_index("c")
    s = lax.axis_index("s")
    base_row = s * ROWS_PER_TILE
    ebase = s * EDGES_PER_TILE
    dummyv = NQ + s * 8 + jnp.bitwise_and(lax.iota(jnp.int32, 16), 7)

    pltpu.sync_copy(zden_hbm, den_loc)
    pltpu.sync_copy(u_hbm, u_loc)

    for phase in range(NPHASE):
        first = phase == 0
        quarter = phase * NCORE + c
        qlo = quarter * NQ

        pltpu.sync_copy(zrows_hbm, acc.at[pl.ds(base_row, ROWS_PER_TILE)])
        plsc.subcore_barrier()

        def process_staged_chunk(qlo, first):
            gcopy = pltpu.async_copy(hu_hbm.at[didx], rows, gsem)
            for g in range(CHUNK // 16):
                sv = sidx[pl.ds(g * 16, 16)]
                if first:
                    dv = didx[pl.ds(g * 16, 16)]
                    uv = plsc.load_gather(u_loc, [dv])
                    plsc.addupdate_scatter(
                        den_loc,
                        [jnp.right_shift(sv, 4), jnp.bitwise_and(sv, 15)],
                        uv)
                sloc = sv - qlo
                ok = (sloc >= 0) & (sloc < NQ)
                sidx[pl.ds(g * 16, 16)] = jnp.where(ok, sloc, dummyv)
            gcopy.wait()
            pltpu.sync_copy(rows, acc.at[sidx], add=True)

        def chunk_body(j, carry):
            off = ebase + j * CHUNK
            pltpu.sync_copy(src_hbm.at[pl.ds(off, CHUNK)], sidx)
            pltpu.sync_copy(dst_hbm.at[pl.ds(off, CHUNK)], didx)
            process_staged_chunk(qlo, first)
            return carry

        lax.fori_loop(0, FULL_CHUNKS, chunk_body, 0)

        toff = ebase + FULL_CHUNKS * CHUNK
        pltpu.sync_copy(src_hbm.at[pl.ds(toff, TAIL)],
                        sidx.at[pl.ds(0, TAIL)])
        pltpu.sync_copy(dst_hbm.at[pl.ds(toff, TAIL)],
                        didx.at[pl.ds(0, TAIL)])
        for i in range(TAIL, CHUNK, 16):
            sidx[pl.ds(i, 16)] = jnp.full((16,), N, jnp.int32)
            didx[pl.ds(i, 16)] = jnp.zeros((16,), jnp.int32)
        process_staged_chunk(qlo, first)

        plsc.subcore_barrier()
        pltpu.sync_copy(acc.at[pl.ds(base_row, ROWS_PER_TILE)],
                        acc_out.at[quarter].at[pl.ds(base_row,
                                                     ROWS_PER_TILE)])

    pltpu.sync_copy(den_loc, den_out.at[c].at[s])


def kernel(x, edge_index, W0, b0, aW0, ab0, W1, b1, aW1, ab1, W2, b2, aW2, ab2):
    src = edge_index[0]
    dst = edge_index[1]
    zrows = jnp.zeros((ROWS_PER_TILE, D), jnp.float32)
    zden = jnp.zeros((DEN_R, DEN_C), jnp.float32)
    xp = jnp.pad(x, ((0, NP - N), (0, 0)))

    Ws = jnp.stack([W0, W1, W2])
    bs = jnp.stack([b0.reshape(1, D), b1.reshape(1, D), b2.reshape(1, D)])
    awds = jnp.stack([aW0[D:], aW1[D:], aW2[D:]])
    abs_ = jnp.stack([ab0.reshape(1, 1), ab1.reshape(1, 1), ab2.reshape(1, 1)])

    def layer_step(li, carry):
        acc, den = carry
        flag = (li == 0).astype(jnp.int32).reshape(1, 1)
        W = lax.dynamic_index_in_dim(Ws, li, keepdims=False)
        b = lax.dynamic_index_in_dim(bs, li, keepdims=False)
        awd = lax.dynamic_index_in_dim(awds, li, keepdims=False)
        ab = lax.dynamic_index_in_dim(abs_, li, keepdims=False)
        hu, u = _tc_layer(flag, xp, acc, den, W, b, awd, ab)
        accq, denq = _sc_segsum(hu, u.reshape(NP), src, dst, zrows, zden)
        den = (denq.sum(axis=(0, 1)) * 0.5).reshape(NP, 1)
        return accq.reshape(NP, D), den

    acc0 = jnp.zeros((NP, D), jnp.float32)
    den0 = jnp.zeros((NP, 1), jnp.float32)
    acc, den = lax.fori_loop(0, 3, layer_step, (acc0, den0))
    return _tc_final(acc, den)[:N]
